# baseline (speedup 1.0000x reference)
import jax
import jax.numpy as jnp
from jax import lax
from jax.experimental import pallas as pl
from jax.experimental.pallas import tpu as pltpu
from jax.experimental.pallas import tpu_sc as plsc

L = 16
NC, NS = 2, 16
NW = NC * NS

GRIDS = ((65, 257, 257), (33, 129, 129))
ED = 16
NCORNERS = 8
OD = 2 * ED
NR = NCORNERS * L


def _corner_idx_weights(x, y, z, grid):
    T, H, W = grid
    tx = x * float(T - 1)
    ty = y * float(H - 1)
    tz = z * float(W - 1)
    ix = tx.astype(jnp.int32)
    iy = ty.astype(jnp.int32)
    iz = tz.astype(jnp.int32)
    rx = tx - ix.astype(jnp.float32)
    ry = ty - iy.astype(jnp.float32)
    rz = tz - iz.astype(jnp.float32)
    ixp = jnp.minimum(ix + 1, T - 1)
    iyp = jnp.minimum(iy + 1, H - 1)
    izp = jnp.minimum(iz + 1, W - 1)
    u0 = ix * (H * W)
    u1 = ixp * (H * W)
    v0 = iy * W
    v1 = iyp * W
    idxs, wts = [], []
    for u, wx in ((u0, 1.0 - rx), (u1, rx)):
        for v, wy in ((v0, 1.0 - ry), (v1, ry)):
            uv = u + v
            wxy = wx * wy
            for zz, wz in ((iz, 1.0 - rz), (izp, rz)):
                idxs.append(uv + zz)
                wts.append(wxy * wz)
    return idxs, wts


def _make_body(n):
    pt = n // NW
    ngroups = pt // L
    assert ngroups % 2 == 0

    nchunks = 4
    gpc = ngroups // nchunks

    def body(inp, t0, t1, out,
             xin_v,
             idx0A, idx1A, wA, rows0A, rows1A, outbA,
             idx0B, idx1B, wB, rows0B, rows1B, outbB,
             gsemA, gsemB, osemA, osemB):
        wid = lax.axis_index("s") * NC + lax.axis_index("c")
        base = wid * pt
        pltpu.sync_copy(inp.at[pl.ds(base * 3, gpc * L * 3)], xin_v)
        lanes = lax.iota(jnp.int32, L)

        def prep(g, idx0, idx1, w):
            r3 = (lax.rem(g, gpc) * L + lanes) * 3
            x = plsc.load_gather(xin_v, [r3])
            y = plsc.load_gather(xin_v, [r3 + 1])
            z = plsc.load_gather(xin_v, [r3 + 2])
            for lvl, idx_v in ((0, idx0), (1, idx1)):
                idxs, wts = _corner_idx_weights(x, y, z, GRIDS[lvl])
                for c in range(NCORNERS):
                    idx_v[pl.ds(c * L, L)] = idxs[c]
                    w[pl.ds(L + (lvl * NCORNERS + c) * L, L)] = wts[c]

        def fire(idx0, idx1, rows0, rows1, sem):
            pltpu.async_copy(t0.at[idx0], rows0, sem)
            pltpu.async_copy(t1.at[idx1], rows1, sem)

        def drain_gather(idx0, idx1, rows0, rows1, sem):
            pltpu.make_async_copy(t0.at[idx0], rows0, sem).wait()
            pltpu.make_async_copy(t1.at[idx1], rows1, sem).wait()

        def drain_out(g, outb, osem):
            @pl.when(g >= 2)
            def _():
                pltpu.make_async_copy(
                    outb, out.at[pl.ds(base * OD, L * OD)], osem).wait()

        def consume(g, w, rows0, rows1, outb, osem):
            drain_out(g, outb, osem)
            for p in range(L):
                for lvl, rows_v in enumerate((rows0, rows1)):
                    acc = None
                    for ci in range(NCORNERS):
                        r = ci * L + p
                        wb = plsc.load_gather(
                            w, [jnp.full((L,), L + lvl * NR + r, jnp.int32)])
                        contrib = wb * rows_v[r, :]
                        acc = contrib if acc is None else acc + contrib
                    outb[pl.ds(p * OD + lvl * ED, ED)] = acc
            pltpu.async_copy(
                outb, out.at[pl.ds((base + g * L) * OD, L * OD)], osem)

        prep(0, idx0A, idx1A, wA)
        fire(idx0A, idx1A, rows0A, rows1A, gsemA)

        @pl.loop(0, ngroups, step=2)
        def _grp(g):
            prep(g + 1, idx0B, idx1B, wB)
            fire(idx0B, idx1B, rows0B, rows1B, gsemB)
            drain_gather(idx0A, idx1A, rows0A, rows1A, gsemA)
            consume(g, wA, rows0A, rows1A, outbA, osemA)

            @pl.when(jnp.logical_and(g + 2 < ngroups,
                                     lax.rem(g + 2, gpc) == 0))
            def _():
                pltpu.sync_copy(
                    inp.at[pl.ds((base + (g + 2) * L) * 3, gpc * L * 3)],
                    xin_v)

            @pl.when(g + 2 < ngroups)
            def _():
                prep(g + 2, idx0A, idx1A, wA)
                fire(idx0A, idx1A, rows0A, rows1A, gsemA)

            drain_gather(idx0B, idx1B, rows0B, rows1B, gsemB)
            consume(g + 1, wB, rows0B, rows1B, outbB, osemB)

        pltpu.make_async_copy(
            outbA, out.at[pl.ds(base * OD, L * OD)], osemA).wait()
        pltpu.make_async_copy(
            outbB, out.at[pl.ds(base * OD, L * OD)], osemB).wait()

    return body


def kernel(inputs, emb0, emb1):
    n = inputs.shape[0]
    t0 = emb0.reshape(-1, ED)
    t1 = emb1.reshape(-1, ED)
    pt = n // NW
    mesh = plsc.VectorSubcoreMesh(core_axis_name="c", subcore_axis_name="s")
    buf_set = [
        pltpu.VMEM((NR,), jnp.int32),
        pltpu.VMEM((NR,), jnp.int32),
        pltpu.VMEM((L + 2 * NR,), jnp.float32),
        pltpu.VMEM((NR, ED), jnp.float32),
        pltpu.VMEM((NR, ED), jnp.float32),
        pltpu.VMEM((L * OD,), jnp.float32),
    ]
    run = pl.kernel(
        _make_body(n),
        out_type=jax.ShapeDtypeStruct((n * OD,), jnp.float32),
        mesh=mesh,
        compiler_params=pltpu.CompilerParams(
            needs_layout_passes=False, use_tc_tiling_on_sc=False),
        scratch_types=[
            pltpu.VMEM((pt // 4 * 3,), jnp.float32),
            *buf_set,
            *buf_set,
            pltpu.SemaphoreType.DMA,
            pltpu.SemaphoreType.DMA,
            pltpu.SemaphoreType.DMA,
            pltpu.SemaphoreType.DMA,
        ],
    )
    flat = run(inputs.reshape(-1), t0, t1)
    return flat.reshape(n, OD)


# speedup vs baseline: 4.8578x; 4.8578x over previous
import jax
import jax.numpy as jnp
from jax import lax
from jax.experimental import pallas as pl
from jax.experimental.pallas import tpu as pltpu
from jax.experimental.pallas import tpu_sc as plsc

L = 16
NC, NS = 2, 16
NW = NC * NS

GRIDS = ((65, 257, 257), (33, 129, 129))
ED = 16
NCORNERS = 8
OD = 2 * ED
NR = NCORNERS * L


def _corner_idx_weights(x, y, z, grid):
    T, H, W = grid
    tx = x * float(T - 1)
    ty = y * float(H - 1)
    tz = z * float(W - 1)
    ix = tx.astype(jnp.int32)
    iy = ty.astype(jnp.int32)
    iz = tz.astype(jnp.int32)
    rx = tx - ix.astype(jnp.float32)
    ry = ty - iy.astype(jnp.float32)
    rz = tz - iz.astype(jnp.float32)
    ixp = jnp.minimum(ix + 1, T - 1)
    iyp = jnp.minimum(iy + 1, H - 1)
    izp = jnp.minimum(iz + 1, W - 1)
    u0 = ix * (H * W)
    u1 = ixp * (H * W)
    v0 = iy * W
    v1 = iyp * W
    idxs, wts = [], []
    for u, wx in ((u0, 1.0 - rx), (u1, rx)):
        for v, wy in ((v0, 1.0 - ry), (v1, ry)):
            uv = u + v
            wxy = wx * wy
            for zz, wz in ((iz, 1.0 - rz), (izp, rz)):
                idxs.append(uv + zz)
                wts.append(wxy * wz)
    return idxs, wts


def _make_body(n):
    pt = n // NW
    ngroups = pt // L
    assert ngroups % 2 == 0

    nchunks = 4
    gpc = ngroups // nchunks

    def body(inp, t0, t1, out,
             xin_v,
             idx0A, idx1A, wA, rows0A, rows1A, outbA,
             idx0B, idx1B, wB, rows0B, rows1B, outbB,
             gsemA, gsemB, osemA, osemB):
        wid = lax.axis_index("s") * NC + lax.axis_index("c")
        base = wid * pt
        pltpu.sync_copy(inp.at[pl.ds(base * 3, gpc * L * 3)], xin_v)
        lanes = lax.iota(jnp.int32, L)

        def prep(g, idx0, idx1, w):
            r3 = (lax.rem(g, gpc) * L + lanes) * 3
            x = plsc.load_gather(xin_v, [r3])
            y = plsc.load_gather(xin_v, [r3 + 1])
            z = plsc.load_gather(xin_v, [r3 + 2])
            for lvl, idx_v in ((0, idx0), (1, idx1)):
                idxs, wts = _corner_idx_weights(x, y, z, GRIDS[lvl])
                for c in range(NCORNERS):
                    idx_v[pl.ds(c * L, L)] = idxs[c]
                    w[pl.ds(L + (lvl * NCORNERS + c) * L, L)] = wts[c]

        def fire(idx0, idx1, rows0, rows1, sem):
            pltpu.async_copy(t0.at[idx0], rows0, sem)
            pltpu.async_copy(t1.at[idx1], rows1, sem)

        def drain_gather(idx0, idx1, rows0, rows1, sem):
            pltpu.make_async_copy(t0.at[idx0], rows0, sem).wait()
            pltpu.make_async_copy(t1.at[idx1], rows1, sem).wait()

        def drain_out(g, outb, osem):
            @pl.when(g >= 2)
            def _():
                pltpu.make_async_copy(
                    outb, out.at[pl.ds(base * OD, L * OD)], osem).wait()

        def consume(g, w, rows0, rows1, outb, osem):
            drain_out(g, outb, osem)
            for p in range(L):
                for lvl, rows_v in enumerate((rows0, rows1)):
                    acc = None
                    for ci in range(NCORNERS):
                        r = ci * L + p
                        wb = plsc.load_gather(
                            w, [jnp.full((L,), L + lvl * NR + r, jnp.int32)])
                        contrib = wb * rows_v[r, :]
                        acc = contrib if acc is None else acc + contrib
                    outb[pl.ds(p * OD + lvl * ED, ED)] = acc
            pltpu.async_copy(
                outb, out.at[pl.ds((base + g * L) * OD, L * OD)], osem)

        prep(0, idx0A, idx1A, wA)
        fire(idx0A, idx1A, rows0A, rows1A, gsemA)

        @pl.loop(0, ngroups, step=2)
        def _grp(g):
            prep(g + 1, idx0B, idx1B, wB)
            fire(idx0B, idx1B, rows0B, rows1B, gsemB)
            drain_gather(idx0A, idx1A, rows0A, rows1A, gsemA)
            consume(g, wA, rows0A, rows1A, outbA, osemA)

            @pl.when(jnp.logical_and(g + 2 < ngroups,
                                     lax.rem(g + 2, gpc) == 0))
            def _():
                pltpu.sync_copy(
                    inp.at[pl.ds((base + (g + 2) * L) * 3, gpc * L * 3)],
                    xin_v)

            @pl.when(g + 2 < ngroups)
            def _():
                prep(g + 2, idx0A, idx1A, wA)
                fire(idx0A, idx1A, rows0A, rows1A, gsemA)

            drain_gather(idx0B, idx1B, rows0B, rows1B, gsemB)
            consume(g + 1, wB, rows0B, rows1B, outbB, osemB)

        pltpu.make_async_copy(
            outbA, out.at[pl.ds(base * OD, L * OD)], osemA).wait()
        pltpu.make_async_copy(
            outbB, out.at[pl.ds(base * OD, L * OD)], osemB).wait()

    return body


def _make_transpose_body():
    levels = []
    for (T, H, W) in GRIDS:
        nblk = T * H
        spt = -(-nblk // NW)
        spt += spt % 2
        levels.append((nblk, spt, W, ED * W))

    def body(f0, f1, o0, o1,
             slabA, slabB, outbA, outbB, isemA, isemB, osemA, osemB):
        wid = lax.axis_index("s") * NC + lax.axis_index("c")
        lanes = lax.iota(jnp.int32, L)
        for lvl, (f, o) in enumerate(((f0, o0), (f1, o1))):
            nblk, spt, W, blkw = levels[lvl]
            civ = lanes * W
            lo = wid * spt

            def fire_in(j, slab, sem):
                pltpu.async_copy(f.at[pl.ds(j * blkw, blkw)],
                                 slab.at[pl.ds(0, blkw)], sem)

            def drain_in(slab, sem):
                pltpu.make_async_copy(f.at[pl.ds(0, blkw)],
                                      slab.at[pl.ds(0, blkw)], sem).wait()

            def transpose(slab, outb):
                @pl.loop(0, W, unroll=4)
                def _w(w):
                    outb[w, :] = plsc.load_gather(slab, [civ + w])

            def fire_out(j, outb, sem):
                pltpu.async_copy(outb.at[pl.ds(0, W), :],
                                 o.at[pl.ds(j * W, W), :], sem)

            def drain_out(outb, sem):
                pltpu.make_async_copy(outb.at[pl.ds(0, W), :],
                                      o.at[pl.ds(0, W), :], sem).wait()

            def jj(j):
                return jnp.minimum(j, nblk - 1)

            fire_in(jj(lo), slabA, isemA)

            @pl.loop(lo, lo + spt, step=2)
            def _blk(g):
                fire_in(jj(g + 1), slabB, isemB)
                drain_in(slabA, isemA)

                @pl.when(g > lo)
                def _():
                    drain_out(outbA, osemA)

                transpose(slabA, outbA)
                fire_out(jj(g), outbA, osemA)

                @pl.when(g + 2 < lo + spt)
                def _():
                    fire_in(jj(g + 2), slabA, isemA)

                drain_in(slabB, isemB)

                @pl.when(g > lo)
                def _():
                    drain_out(outbB, osemB)

                transpose(slabB, outbB)
                fire_out(jj(g + 1), outbB, osemB)

            drain_out(outbA, osemA)
            drain_out(outbB, osemB)

    return body


def _relayout_tables(emb0, emb1):
    mesh = plsc.VectorSubcoreMesh(core_axis_name="c", subcore_axis_name="s")
    maxw = max(W for (_, _, W) in GRIDS)
    run = pl.kernel(
        _make_transpose_body(),
        out_type=(
            jax.ShapeDtypeStruct((GRIDS[0][0] * GRIDS[0][1] * GRIDS[0][2],
                                  ED), jnp.float32),
            jax.ShapeDtypeStruct((GRIDS[1][0] * GRIDS[1][1] * GRIDS[1][2],
                                  ED), jnp.float32),
        ),
        mesh=mesh,
        compiler_params=pltpu.CompilerParams(
            needs_layout_passes=False, use_tc_tiling_on_sc=False),
        scratch_types=[
            pltpu.VMEM((ED * maxw,), jnp.float32),
            pltpu.VMEM((ED * maxw,), jnp.float32),
            pltpu.VMEM((maxw, ED), jnp.float32),
            pltpu.VMEM((maxw, ED), jnp.float32),
            pltpu.SemaphoreType.DMA,
            pltpu.SemaphoreType.DMA,
            pltpu.SemaphoreType.DMA,
            pltpu.SemaphoreType.DMA,
        ],
    )
    f0 = jnp.transpose(emb0, (0, 1, 3, 2)).reshape(-1)
    f1 = jnp.transpose(emb1, (0, 1, 3, 2)).reshape(-1)
    return run(f0, f1)


def kernel(inputs, emb0, emb1):
    n = inputs.shape[0]
    t0, t1 = _relayout_tables(emb0, emb1)
    pt = n // NW
    mesh = plsc.VectorSubcoreMesh(core_axis_name="c", subcore_axis_name="s")
    buf_set = [
        pltpu.VMEM((NR,), jnp.int32),
        pltpu.VMEM((NR,), jnp.int32),
        pltpu.VMEM((L + 2 * NR,), jnp.float32),
        pltpu.VMEM((NR, ED), jnp.float32),
        pltpu.VMEM((NR, ED), jnp.float32),
        pltpu.VMEM((L * OD,), jnp.float32),
    ]
    run = pl.kernel(
        _make_body(n),
        out_type=jax.ShapeDtypeStruct((n * OD,), jnp.float32),
        mesh=mesh,
        compiler_params=pltpu.CompilerParams(
            needs_layout_passes=False, use_tc_tiling_on_sc=False),
        scratch_types=[
            pltpu.VMEM((pt // 4 * 3,), jnp.float32),
            *buf_set,
            *buf_set,
            pltpu.SemaphoreType.DMA,
            pltpu.SemaphoreType.DMA,
            pltpu.SemaphoreType.DMA,
            pltpu.SemaphoreType.DMA,
        ],
    )
    flat = run(inputs.reshape(-1), t0, t1)
    return flat.reshape(n, OD)


# speedup vs baseline: 4.8776x; 1.0041x over previous
import jax
import jax.numpy as jnp
from jax import lax
from jax.experimental import pallas as pl
from jax.experimental.pallas import tpu as pltpu
from jax.experimental.pallas import tpu_sc as plsc

L = 16
NC, NS = 2, 16
NW = NC * NS

GRIDS = ((65, 257, 257), (33, 129, 129))
ED = 16
NCORNERS = 8
OD = 2 * ED
NR = NCORNERS * L


def _corner_idx_weights(x, y, z, grid):
    T, H, W = grid
    tx = x * float(T - 1)
    ty = y * float(H - 1)
    tz = z * float(W - 1)
    ix = tx.astype(jnp.int32)
    iy = ty.astype(jnp.int32)
    iz = tz.astype(jnp.int32)
    rx = tx - ix.astype(jnp.float32)
    ry = ty - iy.astype(jnp.float32)
    rz = tz - iz.astype(jnp.float32)
    ixp = jnp.minimum(ix + 1, T - 1)
    iyp = jnp.minimum(iy + 1, H - 1)
    izp = jnp.minimum(iz + 1, W - 1)
    u0 = ix * (H * W)
    u1 = ixp * (H * W)
    v0 = iy * W
    v1 = iyp * W
    idxs, wts = [], []
    for u, wx in ((u0, 1.0 - rx), (u1, rx)):
        for v, wy in ((v0, 1.0 - ry), (v1, ry)):
            uv = u + v
            wxy = wx * wy
            for zz, wz in ((iz, 1.0 - rz), (izp, rz)):
                idxs.append(uv + zz)
                wts.append(wxy * wz)
    return idxs, wts


def _make_body(n):
    pt = n // NW
    ngroups = pt // L
    assert ngroups % 2 == 0

    nchunks = 4
    gpc = ngroups // nchunks

    def body(inp, t0, t1, out,
             xin_v,
             idx0A, idx1A, wA, rows0A, rows1A, outbA,
             idx0B, idx1B, wB, rows0B, rows1B, outbB,
             gsemA, gsemB, osemA, osemB):
        wid = lax.axis_index("s") * NC + lax.axis_index("c")
        base = wid * pt
        pltpu.sync_copy(inp.at[pl.ds(base * 3, gpc * L * 3)], xin_v)
        lanes = lax.iota(jnp.int32, L)

        def prep(g, idx0, idx1, w):
            r3 = (lax.rem(g, gpc) * L + lanes) * 3
            x = plsc.load_gather(xin_v, [r3])
            y = plsc.load_gather(xin_v, [r3 + 1])
            z = plsc.load_gather(xin_v, [r3 + 2])
            for lvl, idx_v in ((0, idx0), (1, idx1)):
                idxs, wts = _corner_idx_weights(x, y, z, GRIDS[lvl])
                for c in range(NCORNERS):
                    idx_v[pl.ds(c * L, L)] = idxs[c]
                    w[pl.ds(L + (lvl * NCORNERS + c) * L, L)] = wts[c]

        def fire(idx0, idx1, rows0, rows1, sem):
            pltpu.async_copy(t0.at[idx0], rows0, sem)
            pltpu.async_copy(t1.at[idx1], rows1, sem)

        def drain_gather(idx0, idx1, rows0, rows1, sem):
            pltpu.make_async_copy(t0.at[idx0], rows0, sem).wait()
            pltpu.make_async_copy(t1.at[idx1], rows1, sem).wait()

        def drain_out(g, outb, osem):
            @pl.when(g >= 2)
            def _():
                pltpu.make_async_copy(
                    outb, out.at[pl.ds(base, L), :], osem).wait()

        def consume(g, w, rows0, rows1, outb, osem):
            drain_out(g, outb, osem)
            for p in range(L):
                for lvl, rows_v in enumerate((rows0, rows1)):
                    acc = None
                    for ci in range(NCORNERS):
                        r = ci * L + p
                        wb = plsc.load_gather(
                            w, [jnp.full((L,), L + lvl * NR + r, jnp.int32)])
                        contrib = wb * rows_v[r, :]
                        acc = contrib if acc is None else acc + contrib
                    outb[p, pl.ds(lvl * ED, ED)] = acc
            pltpu.async_copy(
                outb, out.at[pl.ds(base + g * L, L), :], osem)

        prep(0, idx0A, idx1A, wA)
        fire(idx0A, idx1A, rows0A, rows1A, gsemA)

        @pl.loop(0, ngroups, step=2)
        def _grp(g):
            prep(g + 1, idx0B, idx1B, wB)
            fire(idx0B, idx1B, rows0B, rows1B, gsemB)
            drain_gather(idx0A, idx1A, rows0A, rows1A, gsemA)
            consume(g, wA, rows0A, rows1A, outbA, osemA)

            @pl.when(jnp.logical_and(g + 2 < ngroups,
                                     lax.rem(g + 2, gpc) == 0))
            def _():
                pltpu.sync_copy(
                    inp.at[pl.ds((base + (g + 2) * L) * 3, gpc * L * 3)],
                    xin_v)

            @pl.when(g + 2 < ngroups)
            def _():
                prep(g + 2, idx0A, idx1A, wA)
                fire(idx0A, idx1A, rows0A, rows1A, gsemA)

            drain_gather(idx0B, idx1B, rows0B, rows1B, gsemB)
            consume(g + 1, wB, rows0B, rows1B, outbB, osemB)

        pltpu.make_async_copy(
            outbA, out.at[pl.ds(base, L), :], osemA).wait()
        pltpu.make_async_copy(
            outbB, out.at[pl.ds(base, L), :], osemB).wait()

    return body


def _make_transpose_body():
    levels = []
    for (T, H, W) in GRIDS:
        nblk = T * H
        spt = -(-nblk // NW)
        spt += spt % 2
        levels.append((nblk, spt, W, ED * W))

    def body(f0, f1, o0, o1,
             slabA, slabB, outbA, outbB, isemA, isemB, osemA, osemB):
        wid = lax.axis_index("s") * NC + lax.axis_index("c")
        lanes = lax.iota(jnp.int32, L)
        for lvl, (f, o) in enumerate(((f0, o0), (f1, o1))):
            nblk, spt, W, blkw = levels[lvl]
            civ = lanes * W
            lo = wid * spt

            def fire_in(j, slab, sem):
                pltpu.async_copy(f.at[pl.ds(j * blkw, blkw)],
                                 slab.at[pl.ds(0, blkw)], sem)

            def drain_in(slab, sem):
                pltpu.make_async_copy(f.at[pl.ds(0, blkw)],
                                      slab.at[pl.ds(0, blkw)], sem).wait()

            def transpose(slab, outb):
                @pl.loop(0, W, unroll=8)
                def _w(w):
                    outb[w, :] = plsc.load_gather(slab, [civ + w])

            def fire_out(j, outb, sem):
                pltpu.async_copy(outb.at[pl.ds(0, W), :],
                                 o.at[pl.ds(j * W, W), :], sem)

            def drain_out(outb, sem):
                pltpu.make_async_copy(outb.at[pl.ds(0, W), :],
                                      o.at[pl.ds(0, W), :], sem).wait()

            def jj(j):
                return jnp.minimum(j, nblk - 1)

            fire_in(jj(lo), slabA, isemA)

            @pl.loop(lo, lo + spt, step=2)
            def _blk(g):
                fire_in(jj(g + 1), slabB, isemB)
                drain_in(slabA, isemA)

                @pl.when(g > lo)
                def _():
                    drain_out(outbA, osemA)

                transpose(slabA, outbA)
                fire_out(jj(g), outbA, osemA)

                @pl.when(g + 2 < lo + spt)
                def _():
                    fire_in(jj(g + 2), slabA, isemA)

                drain_in(slabB, isemB)

                @pl.when(g > lo)
                def _():
                    drain_out(outbB, osemB)

                transpose(slabB, outbB)
                fire_out(jj(g + 1), outbB, osemB)

            drain_out(outbA, osemA)
            drain_out(outbB, osemB)

    return body


def _relayout_tables(emb0, emb1):
    mesh = plsc.VectorSubcoreMesh(core_axis_name="c", subcore_axis_name="s")
    maxw = max(W for (_, _, W) in GRIDS)
    run = pl.kernel(
        _make_transpose_body(),
        out_type=(
            jax.ShapeDtypeStruct((GRIDS[0][0] * GRIDS[0][1] * GRIDS[0][2],
                                  ED), jnp.float32),
            jax.ShapeDtypeStruct((GRIDS[1][0] * GRIDS[1][1] * GRIDS[1][2],
                                  ED), jnp.float32),
        ),
        mesh=mesh,
        compiler_params=pltpu.CompilerParams(
            needs_layout_passes=False, use_tc_tiling_on_sc=False),
        scratch_types=[
            pltpu.VMEM((ED * maxw,), jnp.float32),
            pltpu.VMEM((ED * maxw,), jnp.float32),
            pltpu.VMEM((maxw, ED), jnp.float32),
            pltpu.VMEM((maxw, ED), jnp.float32),
            pltpu.SemaphoreType.DMA,
            pltpu.SemaphoreType.DMA,
            pltpu.SemaphoreType.DMA,
            pltpu.SemaphoreType.DMA,
        ],
    )
    f0 = jnp.transpose(emb0, (0, 1, 3, 2)).reshape(-1)
    f1 = jnp.transpose(emb1, (0, 1, 3, 2)).reshape(-1)
    return run(f0, f1)


def kernel(inputs, emb0, emb1):
    n = inputs.shape[0]
    t0, t1 = _relayout_tables(emb0, emb1)
    pt = n // NW
    mesh = plsc.VectorSubcoreMesh(core_axis_name="c", subcore_axis_name="s")
    buf_set = [
        pltpu.VMEM((NR,), jnp.int32),
        pltpu.VMEM((NR,), jnp.int32),
        pltpu.VMEM((L + 2 * NR,), jnp.float32),
        pltpu.VMEM((NR, ED), jnp.float32),
        pltpu.VMEM((NR, ED), jnp.float32),
        pltpu.VMEM((L, OD), jnp.float32),
    ]
    run = pl.kernel(
        _make_body(n),
        out_type=jax.ShapeDtypeStruct((n, OD), jnp.float32),
        mesh=mesh,
        compiler_params=pltpu.CompilerParams(
            needs_layout_passes=False, use_tc_tiling_on_sc=False),
        scratch_types=[
            pltpu.VMEM((pt // 4 * 3,), jnp.float32),
            *buf_set,
            *buf_set,
            pltpu.SemaphoreType.DMA,
            pltpu.SemaphoreType.DMA,
            pltpu.SemaphoreType.DMA,
            pltpu.SemaphoreType.DMA,
        ],
    )
    return run(inputs.reshape(-1), t0, t1)


# speedup vs baseline: 6.0894x; 1.2484x over previous
import jax
import jax.numpy as jnp
from jax import lax
from jax.experimental import pallas as pl
from jax.experimental.pallas import tpu as pltpu
from jax.experimental.pallas import tpu_sc as plsc

L = 16
NC, NS = 2, 16
NW = NC * NS

GRIDS = ((65, 257, 257), (33, 129, 129))
ED = 16
NCORNERS = 8
OD = 2 * ED
NR = NCORNERS * L


def _corner_idx_weights(x, y, z, grid):
    T, H, W = grid
    tx = x * float(T - 1)
    ty = y * float(H - 1)
    tz = z * float(W - 1)
    ix = tx.astype(jnp.int32)
    iy = ty.astype(jnp.int32)
    iz = tz.astype(jnp.int32)
    rx = tx - ix.astype(jnp.float32)
    ry = ty - iy.astype(jnp.float32)
    rz = tz - iz.astype(jnp.float32)
    ixp = jnp.minimum(ix + 1, T - 1)
    iyp = jnp.minimum(iy + 1, H - 1)
    izp = jnp.minimum(iz + 1, W - 1)
    u0 = ix * (H * W)
    u1 = ixp * (H * W)
    v0 = iy * W
    v1 = iyp * W
    idxs, wts = [], []
    for u, wx in ((u0, 1.0 - rx), (u1, rx)):
        for v, wy in ((v0, 1.0 - ry), (v1, ry)):
            uv = u + v
            wxy = wx * wy
            for zz, wz in ((iz, 1.0 - rz), (izp, rz)):
                idxs.append(uv + zz)
                wts.append(wxy * wz)
    return idxs, wts


def _make_body(n):
    pt = n // NW
    ngroups = pt // L
    assert ngroups % 2 == 0

    nchunks = 4
    gpc = ngroups // nchunks

    def body(inp, t0, t1, out,
             xin_v,
             idx0A, idx1A, wA, rows0A, rows1A, outbA,
             idx0B, idx1B, wB, rows0B, rows1B, outbB,
             gsemA, gsemB, osemA, osemB):
        wid = lax.axis_index("s") * NC + lax.axis_index("c")
        base = wid * pt
        pltpu.sync_copy(inp.at[pl.ds(base * 3, gpc * L * 3)], xin_v)
        lanes = lax.iota(jnp.int32, L)

        def prep(g, idx0, idx1, w):
            r3 = (lax.rem(g, gpc) * L + lanes) * 3
            x = plsc.load_gather(xin_v, [r3])
            y = plsc.load_gather(xin_v, [r3 + 1])
            z = plsc.load_gather(xin_v, [r3 + 2])
            for lvl, idx_v in ((0, idx0), (1, idx1)):
                idxs, wts = _corner_idx_weights(x, y, z, GRIDS[lvl])
                for c in range(NCORNERS):
                    idx_v[pl.ds(c * L, L)] = idxs[c]
                    w[pl.ds(L + (lvl * NCORNERS + c) * L, L)] = wts[c]

        def fire(idx0, idx1, rows0, rows1, sem):
            pltpu.async_copy(t0.at[idx0], rows0, sem)
            pltpu.async_copy(t1.at[idx1], rows1, sem)

        def drain_gather(idx0, idx1, rows0, rows1, sem):
            pltpu.make_async_copy(t0.at[idx0], rows0, sem).wait()
            pltpu.make_async_copy(t1.at[idx1], rows1, sem).wait()

        def drain_out(g, outb, osem):
            @pl.when(g >= 2)
            def _():
                pltpu.make_async_copy(
                    outb, out.at[pl.ds(base, L), :], osem).wait()

        lanes16 = lanes * L

        def consume(g, w, rows0, rows1, outb, osem):
            drain_out(g, outb, osem)
            for p in range(L):
                wvec = plsc.load_gather(w, [lanes16 + (L + p)])
                for lvl, rows_v in enumerate((rows0, rows1)):
                    acc = None
                    for ci in range(NCORNERS):
                        slot = lvl * NCORNERS + ci
                        if slot == 0:
                            wb = plsc.load_gather(
                                w, [jnp.full((L,), L + p, jnp.int32)])
                        else:
                            wb = wvec.at[
                                jnp.full((L,), slot, jnp.int32)
                            ].get(mode="promise_in_bounds")
                        contrib = wb * rows_v[ci * L + p, :]
                        acc = contrib if acc is None else acc + contrib
                    outb[p, pl.ds(lvl * ED, ED)] = acc
            pltpu.async_copy(
                outb, out.at[pl.ds(base + g * L, L), :], osem)

        prep(0, idx0A, idx1A, wA)
        fire(idx0A, idx1A, rows0A, rows1A, gsemA)

        @pl.loop(0, ngroups, step=2)
        def _grp(g):
            prep(g + 1, idx0B, idx1B, wB)
            fire(idx0B, idx1B, rows0B, rows1B, gsemB)
            drain_gather(idx0A, idx1A, rows0A, rows1A, gsemA)
            consume(g, wA, rows0A, rows1A, outbA, osemA)

            @pl.when(jnp.logical_and(g + 2 < ngroups,
                                     lax.rem(g + 2, gpc) == 0))
            def _():
                pltpu.sync_copy(
                    inp.at[pl.ds((base + (g + 2) * L) * 3, gpc * L * 3)],
                    xin_v)

            @pl.when(g + 2 < ngroups)
            def _():
                prep(g + 2, idx0A, idx1A, wA)
                fire(idx0A, idx1A, rows0A, rows1A, gsemA)

            drain_gather(idx0B, idx1B, rows0B, rows1B, gsemB)
            consume(g + 1, wB, rows0B, rows1B, outbB, osemB)

        pltpu.make_async_copy(
            outbA, out.at[pl.ds(base, L), :], osemA).wait()
        pltpu.make_async_copy(
            outbB, out.at[pl.ds(base, L), :], osemB).wait()

    return body


def _make_transpose_body():
    levels = []
    for (T, H, W) in GRIDS:
        nblk = T * H
        spt = -(-nblk // NW)
        spt += spt % 2
        levels.append((nblk, spt, W, ED * W))

    def body(f0, f1, o0, o1,
             slabA, slabB, outbA, outbB, isemA, isemB, osemA, osemB):
        wid = lax.axis_index("s") * NC + lax.axis_index("c")
        lanes = lax.iota(jnp.int32, L)
        for lvl, (f, o) in enumerate(((f0, o0), (f1, o1))):
            nblk, spt, W, blkw = levels[lvl]
            civ = lanes * W
            lo = wid * spt

            def fire_in(j, slab, sem):
                pltpu.async_copy(f.at[pl.ds(j * blkw, blkw)],
                                 slab.at[pl.ds(0, blkw)], sem)

            def drain_in(slab, sem):
                pltpu.make_async_copy(f.at[pl.ds(0, blkw)],
                                      slab.at[pl.ds(0, blkw)], sem).wait()

            def transpose(slab, outb):
                @pl.loop(0, W, unroll=8)
                def _w(w):
                    outb[w, :] = plsc.load_gather(slab, [civ + w])

            def fire_out(j, outb, sem):
                pltpu.async_copy(outb.at[pl.ds(0, W), :],
                                 o.at[pl.ds(j * W, W), :], sem)

            def drain_out(outb, sem):
                pltpu.make_async_copy(outb.at[pl.ds(0, W), :],
                                      o.at[pl.ds(0, W), :], sem).wait()

            def jj(j):
                return jnp.minimum(j, nblk - 1)

            fire_in(jj(lo), slabA, isemA)

            @pl.loop(lo, lo + spt, step=2)
            def _blk(g):
                fire_in(jj(g + 1), slabB, isemB)
                drain_in(slabA, isemA)

                @pl.when(g > lo)
                def _():
                    drain_out(outbA, osemA)

                transpose(slabA, outbA)
                fire_out(jj(g), outbA, osemA)

                @pl.when(g + 2 < lo + spt)
                def _():
                    fire_in(jj(g + 2), slabA, isemA)

                drain_in(slabB, isemB)

                @pl.when(g > lo)
                def _():
                    drain_out(outbB, osemB)

                transpose(slabB, outbB)
                fire_out(jj(g + 1), outbB, osemB)

            drain_out(outbA, osemA)
            drain_out(outbB, osemB)

    return body


def _relayout_tables(emb0, emb1):
    mesh = plsc.VectorSubcoreMesh(core_axis_name="c", subcore_axis_name="s")
    maxw = max(W for (_, _, W) in GRIDS)
    run = pl.kernel(
        _make_transpose_body(),
        out_type=(
            jax.ShapeDtypeStruct((GRIDS[0][0] * GRIDS[0][1] * GRIDS[0][2],
                                  ED), jnp.float32),
            jax.ShapeDtypeStruct((GRIDS[1][0] * GRIDS[1][1] * GRIDS[1][2],
                                  ED), jnp.float32),
        ),
        mesh=mesh,
        compiler_params=pltpu.CompilerParams(
            needs_layout_passes=False, use_tc_tiling_on_sc=False),
        scratch_types=[
            pltpu.VMEM((ED * maxw,), jnp.float32),
            pltpu.VMEM((ED * maxw,), jnp.float32),
            pltpu.VMEM((maxw, ED), jnp.float32),
            pltpu.VMEM((maxw, ED), jnp.float32),
            pltpu.SemaphoreType.DMA,
            pltpu.SemaphoreType.DMA,
            pltpu.SemaphoreType.DMA,
            pltpu.SemaphoreType.DMA,
        ],
    )
    f0 = jnp.transpose(emb0, (0, 1, 3, 2)).reshape(-1)
    f1 = jnp.transpose(emb1, (0, 1, 3, 2)).reshape(-1)
    return run(f0, f1)


def kernel(inputs, emb0, emb1):
    n = inputs.shape[0]
    t0, t1 = _relayout_tables(emb0, emb1)
    pt = n // NW
    mesh = plsc.VectorSubcoreMesh(core_axis_name="c", subcore_axis_name="s")
    buf_set = [
        pltpu.VMEM((NR,), jnp.int32),
        pltpu.VMEM((NR,), jnp.int32),
        pltpu.VMEM((L + 2 * NR,), jnp.float32),
        pltpu.VMEM((NR, ED), jnp.float32),
        pltpu.VMEM((NR, ED), jnp.float32),
        pltpu.VMEM((L, OD), jnp.float32),
    ]
    run = pl.kernel(
        _make_body(n),
        out_type=jax.ShapeDtypeStruct((n, OD), jnp.float32),
        mesh=mesh,
        compiler_params=pltpu.CompilerParams(
            needs_layout_passes=False, use_tc_tiling_on_sc=False),
        scratch_types=[
            pltpu.VMEM((pt // 4 * 3,), jnp.float32),
            *buf_set,
            *buf_set,
            pltpu.SemaphoreType.DMA,
            pltpu.SemaphoreType.DMA,
            pltpu.SemaphoreType.DMA,
            pltpu.SemaphoreType.DMA,
        ],
    )
    return run(inputs.reshape(-1), t0, t1)


# speedup vs baseline: 6.1390x; 1.0082x over previous
import jax
import jax.numpy as jnp
from jax import lax
from jax.experimental import pallas as pl
from jax.experimental.pallas import tpu as pltpu
from jax.experimental.pallas import tpu_sc as plsc

L = 16
NC, NS = 2, 16
NW = NC * NS

GRIDS = ((65, 257, 257), (33, 129, 129))
ED = 16
NCORNERS = 8
OD = 2 * ED
NR = NCORNERS * L


def _corner_idx_weights(x, y, z, grid):
    T, H, W = grid
    tx = x * float(T - 1)
    ty = y * float(H - 1)
    tz = z * float(W - 1)
    ix = tx.astype(jnp.int32)
    iy = ty.astype(jnp.int32)
    iz = tz.astype(jnp.int32)
    rx = tx - ix.astype(jnp.float32)
    ry = ty - iy.astype(jnp.float32)
    rz = tz - iz.astype(jnp.float32)
    ixp = jnp.minimum(ix + 1, T - 1)
    iyp = jnp.minimum(iy + 1, H - 1)
    izp = jnp.minimum(iz + 1, W - 1)
    u0 = ix * (H * W)
    u1 = ixp * (H * W)
    v0 = iy * W
    v1 = iyp * W
    idxs, wts = [], []
    for u, wx in ((u0, 1.0 - rx), (u1, rx)):
        for v, wy in ((v0, 1.0 - ry), (v1, ry)):
            uv = u + v
            wxy = wx * wy
            for zz, wz in ((iz, 1.0 - rz), (izp, rz)):
                idxs.append(uv + zz)
                wts.append(wxy * wz)
    return idxs, wts


def _make_body(n):
    pt = n // NW
    ngroups = pt // L
    assert ngroups % 4 == 0

    nchunks = 4
    gpc = ngroups // nchunks

    def body(inp, t0, t1, out,
             xin_v,
             idx0A, idx1A, wA, rows0A, rows1A, outbA,
             idx0B, idx1B, wB, rows0B, rows1B, outbB,
             idx0C, idx1C, wC, rows0C, rows1C, outbC,
             idx0D, idx1D, wD, rows0D, rows1D, outbD,
             gsemA, gsemB, gsemC, gsemD, osemA, osemB, osemC, osemD):
        wid = lax.axis_index("s") * NC + lax.axis_index("c")
        base = wid * pt
        pltpu.sync_copy(inp.at[pl.ds(base * 3, gpc * L * 3)], xin_v)
        lanes = lax.iota(jnp.int32, L)

        def prep(g, idx0, idx1, w):
            r3 = (lax.rem(g, gpc) * L + lanes) * 3
            x = plsc.load_gather(xin_v, [r3])
            y = plsc.load_gather(xin_v, [r3 + 1])
            z = plsc.load_gather(xin_v, [r3 + 2])
            for lvl, idx_v in ((0, idx0), (1, idx1)):
                idxs, wts = _corner_idx_weights(x, y, z, GRIDS[lvl])
                for c in range(NCORNERS):
                    idx_v[pl.ds(c * L, L)] = idxs[c]
                    w[pl.ds(L + (lvl * NCORNERS + c) * L, L)] = wts[c]

        def fire(idx0, idx1, rows0, rows1, sem):
            pltpu.async_copy(t0.at[idx0], rows0, sem)
            pltpu.async_copy(t1.at[idx1], rows1, sem)

        def drain_gather(idx0, idx1, rows0, rows1, sem):
            pltpu.make_async_copy(t0.at[idx0], rows0, sem).wait()
            pltpu.make_async_copy(t1.at[idx1], rows1, sem).wait()

        def drain_out(g, outb, osem):
            @pl.when(g >= 4)
            def _():
                pltpu.make_async_copy(
                    outb, out.at[pl.ds(base, L), :], osem).wait()

        lanes16 = lanes * L

        def consume(g, w, rows0, rows1, outb, osem):
            drain_out(g, outb, osem)
            for p in range(L):
                wvec = plsc.load_gather(w, [lanes16 + (L + p)])
                for lvl, rows_v in enumerate((rows0, rows1)):
                    acc = None
                    for ci in range(NCORNERS):
                        slot = lvl * NCORNERS + ci
                        if slot == 0:
                            wb = plsc.load_gather(
                                w, [jnp.full((L,), L + p, jnp.int32)])
                        else:
                            wb = wvec.at[
                                jnp.full((L,), slot, jnp.int32)
                            ].get(mode="promise_in_bounds")
                        contrib = wb * rows_v[ci * L + p, :]
                        acc = contrib if acc is None else acc + contrib
                    outb[p, pl.ds(lvl * ED, ED)] = acc
            pltpu.async_copy(
                outb, out.at[pl.ds(base + g * L, L), :], osem)

        sets = (
            (idx0A, idx1A, wA, rows0A, rows1A, outbA, gsemA, osemA),
            (idx0B, idx1B, wB, rows0B, rows1B, outbB, gsemB, osemB),
            (idx0C, idx1C, wC, rows0C, rows1C, outbC, gsemC, osemC),
            (idx0D, idx1D, wD, rows0D, rows1D, outbD, gsemD, osemD),
        )
        DEPTH = len(sets)

        for k in range(DEPTH - 1):
            s = sets[k]
            prep(k, s[0], s[1], s[2])
            fire(s[0], s[1], s[3], s[4], s[6])

        @pl.loop(0, ngroups, step=DEPTH)
        def _grp(g):
            for k in range(DEPTH):
                gg = g + k
                s = sets[k]
                sn = sets[(k + DEPTH - 1) % DEPTH]

                @pl.when(jnp.logical_and(gg + DEPTH - 1 < ngroups,
                                         lax.rem(gg + DEPTH - 1, gpc) == 0))
                def _():
                    pltpu.sync_copy(
                        inp.at[pl.ds((base + (gg + DEPTH - 1) * L) * 3,
                                     gpc * L * 3)],
                        xin_v)

                @pl.when(gg + DEPTH - 1 < ngroups)
                def _():
                    prep(gg + DEPTH - 1, sn[0], sn[1], sn[2])
                    fire(sn[0], sn[1], sn[3], sn[4], sn[6])

                drain_gather(s[0], s[1], s[3], s[4], s[6])
                consume(gg, s[2], s[3], s[4], s[5], s[7])

        for s in sets:
            pltpu.make_async_copy(
                s[5], out.at[pl.ds(base, L), :], s[7]).wait()

    return body


def _make_transpose_body():
    levels = []
    for (T, H, W) in GRIDS:
        nblk = T * H
        spt = -(-nblk // NW)
        spt += spt % 2
        levels.append((nblk, spt, W, ED * W))

    def body(f0, f1, o0, o1,
             slabA, slabB, outbA, outbB, isemA, isemB, osemA, osemB):
        wid = lax.axis_index("s") * NC + lax.axis_index("c")
        lanes = lax.iota(jnp.int32, L)
        for lvl, (f, o) in enumerate(((f0, o0), (f1, o1))):
            nblk, spt, W, blkw = levels[lvl]
            civ = lanes * W
            lo = wid * spt

            def fire_in(j, slab, sem):
                pltpu.async_copy(f.at[pl.ds(j * blkw, blkw)],
                                 slab.at[pl.ds(0, blkw)], sem)

            def drain_in(slab, sem):
                pltpu.make_async_copy(f.at[pl.ds(0, blkw)],
                                      slab.at[pl.ds(0, blkw)], sem).wait()

            def transpose(slab, outb):
                @pl.loop(0, W, unroll=8)
                def _w(w):
                    outb[w, :] = plsc.load_gather(slab, [civ + w])

            def fire_out(j, outb, sem):
                pltpu.async_copy(outb.at[pl.ds(0, W), :],
                                 o.at[pl.ds(j * W, W), :], sem)

            def drain_out(outb, sem):
                pltpu.make_async_copy(outb.at[pl.ds(0, W), :],
                                      o.at[pl.ds(0, W), :], sem).wait()

            def jj(j):
                return jnp.minimum(j, nblk - 1)

            fire_in(jj(lo), slabA, isemA)

            @pl.loop(lo, lo + spt, step=2)
            def _blk(g):
                fire_in(jj(g + 1), slabB, isemB)
                drain_in(slabA, isemA)

                @pl.when(g > lo)
                def _():
                    drain_out(outbA, osemA)

                transpose(slabA, outbA)
                fire_out(jj(g), outbA, osemA)

                @pl.when(g + 2 < lo + spt)
                def _():
                    fire_in(jj(g + 2), slabA, isemA)

                drain_in(slabB, isemB)

                @pl.when(g > lo)
                def _():
                    drain_out(outbB, osemB)

                transpose(slabB, outbB)
                fire_out(jj(g + 1), outbB, osemB)

            drain_out(outbA, osemA)
            drain_out(outbB, osemB)

    return body


def _relayout_tables(emb0, emb1):
    mesh = plsc.VectorSubcoreMesh(core_axis_name="c", subcore_axis_name="s")
    maxw = max(W for (_, _, W) in GRIDS)
    run = pl.kernel(
        _make_transpose_body(),
        out_type=(
            jax.ShapeDtypeStruct((GRIDS[0][0] * GRIDS[0][1] * GRIDS[0][2],
                                  ED), jnp.float32),
            jax.ShapeDtypeStruct((GRIDS[1][0] * GRIDS[1][1] * GRIDS[1][2],
                                  ED), jnp.float32),
        ),
        mesh=mesh,
        compiler_params=pltpu.CompilerParams(
            needs_layout_passes=False, use_tc_tiling_on_sc=False),
        scratch_types=[
            pltpu.VMEM((ED * maxw,), jnp.float32),
            pltpu.VMEM((ED * maxw,), jnp.float32),
            pltpu.VMEM((maxw, ED), jnp.float32),
            pltpu.VMEM((maxw, ED), jnp.float32),
            pltpu.SemaphoreType.DMA,
            pltpu.SemaphoreType.DMA,
            pltpu.SemaphoreType.DMA,
            pltpu.SemaphoreType.DMA,
        ],
    )
    f0 = jnp.transpose(emb0, (0, 1, 3, 2)).reshape(-1)
    f1 = jnp.transpose(emb1, (0, 1, 3, 2)).reshape(-1)
    return run(f0, f1)


def kernel(inputs, emb0, emb1):
    n = inputs.shape[0]
    t0, t1 = _relayout_tables(emb0, emb1)
    pt = n // NW
    mesh = plsc.VectorSubcoreMesh(core_axis_name="c", subcore_axis_name="s")
    buf_set = [
        pltpu.VMEM((NR,), jnp.int32),
        pltpu.VMEM((NR,), jnp.int32),
        pltpu.VMEM((L + 2 * NR,), jnp.float32),
        pltpu.VMEM((NR, ED), jnp.float32),
        pltpu.VMEM((NR, ED), jnp.float32),
        pltpu.VMEM((L, OD), jnp.float32),
    ]
    run = pl.kernel(
        _make_body(n),
        out_type=jax.ShapeDtypeStruct((n, OD), jnp.float32),
        mesh=mesh,
        compiler_params=pltpu.CompilerParams(
            needs_layout_passes=False, use_tc_tiling_on_sc=False),
        scratch_types=[
            pltpu.VMEM((pt // 4 * 3,), jnp.float32),
            *buf_set,
            *buf_set,
            *buf_set,
            *buf_set,
            pltpu.SemaphoreType.DMA,
            pltpu.SemaphoreType.DMA,
            pltpu.SemaphoreType.DMA,
            pltpu.SemaphoreType.DMA,
            pltpu.SemaphoreType.DMA,
            pltpu.SemaphoreType.DMA,
            pltpu.SemaphoreType.DMA,
            pltpu.SemaphoreType.DMA,
        ],
    )
    return run(inputs.reshape(-1), t0, t1)
